# Initial kernel scaffold; baseline (speedup 1.0000x reference)
import jax
import jax.numpy as jnp
from jax.experimental import pallas as pl


def kernel(in1, in2):
    raise NotImplementedError("write your pallas kernel here")


# speedup vs baseline: 10.0595x; 10.0595x over previous
import math
from fractions import Fraction

import numpy as np
import jax
import jax.numpy as jnp
from jax.experimental import pallas as pl
from jax.experimental.pallas import tpu as pltpu

_IRREPS = [(16, 0, 1), (16, 1, -1), (16, 2, 1), (16, 3, -1), (16, 4, 1)]
_BATCH = 128
_NCOLS = 160000


def _fact(n):
    return math.factorial(round(n))


def _su2_cg_coeff(j1, m1, j2, m2, j3, m3):
    if m3 != m1 + m2:
        return 0.0
    vmin = int(max(-j1 + j2 + m3, -j1 + m1, 0))
    vmax = int(min(j2 + j3 + m1, j3 - j1 + j2, j3 + m3))
    C = ((2.0 * j3 + 1.0) * Fraction(
        _fact(j3 + j1 - j2) * _fact(j3 - j1 + j2) * _fact(j1 + j2 - j3)
        * _fact(j3 + m3) * _fact(j3 - m3),
        _fact(j1 + j2 + j3 + 1) * _fact(j1 - m1) * _fact(j1 + m1)
        * _fact(j2 - m2) * _fact(j2 + m2))) ** 0.5
    S = 0
    for v in range(vmin, vmax + 1):
        S += (-1) ** int(v + j2 + m2) * Fraction(
            _fact(j2 + j3 + m1 - v) * _fact(j1 - m1 + v),
            _fact(v) * _fact(j3 - j1 + j2 - v) * _fact(j3 + m3 - v)
            * _fact(j1 - j2 - m3 + v))
    return float(C * S)


def _su2_cg(j1, j2, j3):
    mat = np.zeros((2 * j1 + 1, 2 * j2 + 1, 2 * j3 + 1))
    for m1 in range(-j1, j1 + 1):
        for m2 in range(-j2, j2 + 1):
            m3 = m1 + m2
            if abs(m3) <= j3:
                mat[j1 + m1, j2 + m2, j3 + m3] = _su2_cg_coeff(j1, m1, j2, m2, j3, m3)
    return mat


def _real_basis(l):
    q = np.zeros((2 * l + 1, 2 * l + 1), dtype=np.complex128)
    for m in range(-l, 0):
        q[l + m, l + abs(m)] = 1.0 / 2 ** 0.5
        q[l + m, l - abs(m)] = -1j / 2 ** 0.5
    q[l, l] = 1.0
    for m in range(1, l + 1):
        q[l + m, l + abs(m)] = (-1) ** m / 2 ** 0.5
        q[l + m, l - abs(m)] = 1j * (-1) ** m / 2 ** 0.5
    return (-1j) ** l * q


def _w3j(l1, l2, l3):
    C = _su2_cg(l1, l2, l3).astype(np.complex128)
    Q1 = _real_basis(l1)
    Q2 = _real_basis(l2)
    Q3 = _real_basis(l3)
    C = np.einsum('ij,kl,mn,ikn->jlm', Q1, Q2, np.conj(Q3.T), C)
    C = np.real(C)
    n = np.linalg.norm(C)
    if n > 0:
        C = C / n
    return C


def _build_plan():
    offs = []
    o = 0
    for (m, l, p) in _IRREPS:
        offs.append(o)
        o += m * (2 * l + 1)
    gen = []
    for a, (m1, l1, p1) in enumerate(_IRREPS):
        for b, (m2, l2, p2) in enumerate(_IRREPS):
            for l3 in range(abs(l1 - l2), l1 + l2 + 1):
                p3 = int((p1 * p2 + 1) / 2)
                C = (_w3j(l1, l2, l3) * math.sqrt(2 * l3 + 1)).astype(np.float32)
                gen.append(dict(o1=offs[a], o2=offs[b],
                                I=2 * l1 + 1, J=2 * l2 + 1, K=2 * l3 + 1,
                                key=2 * (l3 + 1) + p3, C=C))
    order = sorted(range(len(gen)), key=lambda i: gen[i]['key'])
    soff = 0
    plan = []
    for gi in order:
        t = dict(gen[gi])
        t['soff'] = soff
        soff += 256 * t['K']
        C = t['C']
        inz = []
        for i in range(t['I']):
            inz.append([(j, k, float(C[i, j, k]))
                        for j in range(t['J']) for k in range(t['K'])
                        if abs(C[i, j, k]) > 0.0])
        t['inz'] = inz
        plan.append(t)
    assert soff == _NCOLS
    return plan


_PLAN = _build_plan()
_MAXSEG = 256 * max(t['K'] for t in _PLAN)


def _row_perms():
    perm = []
    o = 0
    for (m, l, p) in _IRREPS:
        d = 2 * l + 1
        for i in range(d):
            for u in range(m):
                perm.append(o + u * d + i)
        o += m * d
    return np.asarray(perm, np.int32)


_PERM = _row_perms()


def _build_ctab():
    cols = []
    for t in _PLAN:
        t['jcols'] = []
        for i in range(t['I']):
            row = []
            if t['inz'][i]:
                for j in range(t['J']):
                    cvec = t['C'][i, j, :]
                    if np.any(cvec):
                        row.append((j, len(cols)))
                        v = np.zeros(24, np.float32)
                        v[:t['K']] = cvec
                        cols.append(v)
            t['jcols'].append(row)
    n = len(cols)
    npad = -n % 128
    tab = np.stack(cols + [np.zeros(24, np.float32)] * npad, axis=1)
    return tab


_CTAB = _build_ctab()


def _tp_kernel(x1_ref, x2_ref, ct_ref, out_ref, ob0, ob1, sems):
    bufs = (ob0, ob1)
    pending = [None, None]
    for idx, t in enumerate(_PLAN):
        I, J, K = t['I'], t['J'], t['K']
        seg = 16 * K
        buf = idx % 2
        ob = bufs[buf]
        if pending[buf] is not None:
            pending[buf].wait()
            pending[buf] = None
        o1, o2 = t['o1'], t['o2']
        acc = None
        for i in range(I):
            if not t['inz'][i]:
                continue
            w3 = None
            for (j, col) in t['jcols'][i]:
                x2j = x2_ref[o2 + 16 * j:o2 + 16 * (j + 1), :]
                cv = ct_ref[0:K, col:col + 1]
                term = x2j[:, None, :] * cv[None, :, :]
                w3 = term if w3 is None else w3 + term
            w = w3.reshape(seg, _BATCH)
            x1i = x1_ref[o1 + 16 * i:o1 + 16 * (i + 1), :]
            contrib = x1i[:, None, :] * w[None, :, :]
            acc = contrib if acc is None else acc + contrib
        if acc is None:
            acc = jnp.zeros((16, seg, _BATCH), jnp.float32)
        for u in range(16):
            ob[:, u * seg:(u + 1) * seg] = acc[u].T
        cp = pltpu.make_async_copy(
            ob.at[:, 0:256 * K],
            out_ref.at[:, t['soff']:t['soff'] + 256 * K],
            sems.at[buf])
        cp.start()
        pending[buf] = cp
    for p in pending:
        if p is not None:
            p.wait()


@jax.jit
def kernel(in1, in2):
    perm = jnp.asarray(_PERM)
    x1t = in1.T[perm]
    x2t = in2.T[perm]
    return pl.pallas_call(
        _tp_kernel,
        out_shape=jax.ShapeDtypeStruct((_BATCH, _NCOLS), jnp.float32),
        in_specs=[
            pl.BlockSpec(memory_space=pltpu.MemorySpace.VMEM),
            pl.BlockSpec(memory_space=pltpu.MemorySpace.VMEM),
            pl.BlockSpec(memory_space=pltpu.MemorySpace.VMEM),
        ],
        out_specs=pl.BlockSpec(memory_space=pltpu.MemorySpace.HBM),
        scratch_shapes=[
            pltpu.VMEM((_BATCH, _MAXSEG), jnp.float32),
            pltpu.VMEM((_BATCH, _MAXSEG), jnp.float32),
            pltpu.SemaphoreType.DMA((2,)),
        ],
    )(x1t, x2t, jnp.asarray(_CTAB))


# speedup vs baseline: 13.1106x; 1.3033x over previous
import math
from fractions import Fraction

import numpy as np
import jax
import jax.numpy as jnp
from jax.experimental import pallas as pl
from jax.experimental.pallas import tpu as pltpu

_IRREPS = [(16, 0, 1), (16, 1, -1), (16, 2, 1), (16, 3, -1), (16, 4, 1)]
_BATCH = 128
_NCOLS = 160000


def _fact(n):
    return math.factorial(round(n))


def _su2_cg_coeff(j1, m1, j2, m2, j3, m3):
    if m3 != m1 + m2:
        return 0.0
    vmin = int(max(-j1 + j2 + m3, -j1 + m1, 0))
    vmax = int(min(j2 + j3 + m1, j3 - j1 + j2, j3 + m3))
    C = ((2.0 * j3 + 1.0) * Fraction(
        _fact(j3 + j1 - j2) * _fact(j3 - j1 + j2) * _fact(j1 + j2 - j3)
        * _fact(j3 + m3) * _fact(j3 - m3),
        _fact(j1 + j2 + j3 + 1) * _fact(j1 - m1) * _fact(j1 + m1)
        * _fact(j2 - m2) * _fact(j2 + m2))) ** 0.5
    S = 0
    for v in range(vmin, vmax + 1):
        S += (-1) ** int(v + j2 + m2) * Fraction(
            _fact(j2 + j3 + m1 - v) * _fact(j1 - m1 + v),
            _fact(v) * _fact(j3 - j1 + j2 - v) * _fact(j3 + m3 - v)
            * _fact(j1 - j2 - m3 + v))
    return float(C * S)


def _su2_cg(j1, j2, j3):
    mat = np.zeros((2 * j1 + 1, 2 * j2 + 1, 2 * j3 + 1))
    for m1 in range(-j1, j1 + 1):
        for m2 in range(-j2, j2 + 1):
            m3 = m1 + m2
            if abs(m3) <= j3:
                mat[j1 + m1, j2 + m2, j3 + m3] = _su2_cg_coeff(j1, m1, j2, m2, j3, m3)
    return mat


def _real_basis(l):
    q = np.zeros((2 * l + 1, 2 * l + 1), dtype=np.complex128)
    for m in range(-l, 0):
        q[l + m, l + abs(m)] = 1.0 / 2 ** 0.5
        q[l + m, l - abs(m)] = -1j / 2 ** 0.5
    q[l, l] = 1.0
    for m in range(1, l + 1):
        q[l + m, l + abs(m)] = (-1) ** m / 2 ** 0.5
        q[l + m, l - abs(m)] = 1j * (-1) ** m / 2 ** 0.5
    return (-1j) ** l * q


def _w3j(l1, l2, l3):
    C = _su2_cg(l1, l2, l3).astype(np.complex128)
    Q1 = _real_basis(l1)
    Q2 = _real_basis(l2)
    Q3 = _real_basis(l3)
    C = np.einsum('ij,kl,mn,ikn->jlm', Q1, Q2, np.conj(Q3.T), C)
    C = np.real(C)
    n = np.linalg.norm(C)
    if n > 0:
        C = C / n
    return C


def _build_plan():
    offs = []
    o = 0
    for (m, l, p) in _IRREPS:
        offs.append(o)
        o += m * (2 * l + 1)
    gen = []
    for a, (m1, l1, p1) in enumerate(_IRREPS):
        for b, (m2, l2, p2) in enumerate(_IRREPS):
            for l3 in range(abs(l1 - l2), l1 + l2 + 1):
                p3 = int((p1 * p2 + 1) / 2)
                C = (_w3j(l1, l2, l3) * math.sqrt(2 * l3 + 1)).astype(np.float32)
                gen.append(dict(o1=offs[a], o2=offs[b],
                                I=2 * l1 + 1, J=2 * l2 + 1, K=2 * l3 + 1,
                                key=2 * (l3 + 1) + p3, C=C))
    order = sorted(range(len(gen)), key=lambda i: gen[i]['key'])
    soff = 0
    plan = []
    for gi in order:
        t = dict(gen[gi])
        t['soff'] = soff
        soff += 256 * t['K']
        C = t['C']
        inz = []
        for i in range(t['I']):
            inz.append([(j, k, float(C[i, j, k]))
                        for j in range(t['J']) for k in range(t['K'])
                        if abs(C[i, j, k]) > 0.0])
        t['inz'] = inz
        plan.append(t)
    assert soff == _NCOLS
    return plan


_PLAN = _build_plan()
_MAXSEG = 256 * max(t['K'] for t in _PLAN)


def _row_perms():
    perm = []
    o = 0
    for (m, l, p) in _IRREPS:
        d = 2 * l + 1
        for i in range(d):
            for u in range(m):
                perm.append(o + u * d + i)
        o += m * d
    return np.asarray(perm, np.int32)


_PERM = _row_perms()


def _tp_kernel(x1_ref, x2_ref, out_ref, ob0, ob1, sems):
    bufs = (ob0, ob1)
    pending = [None, None]
    for idx, t in enumerate(_PLAN):
        I, J, K = t['I'], t['J'], t['K']
        seg = 16 * K
        buf = idx % 2
        ob = bufs[buf]
        if pending[buf] is not None:
            pending[buf].wait()
            pending[buf] = None
        o1, o2 = t['o1'], t['o2']
        x2j_cache = {}
        acc = None
        for i in range(I):
            if not t['inz'][i]:
                continue
            by_k = {}
            for (j, k, c) in t['inz'][i]:
                if j not in x2j_cache:
                    x2j_cache[j] = x2_ref[o2 + 16 * j:o2 + 16 * (j + 1), :]
                term = x2j_cache[j] * np.float32(c)
                by_k[k] = term if k not in by_k else by_k[k] + term
            wk = [by_k.get(k) for k in range(K)]
            zero = None
            for k in range(K):
                if wk[k] is None:
                    if zero is None:
                        zero = jnp.zeros((16, _BATCH), jnp.float32)
                    wk[k] = zero
            w = jnp.stack(wk, axis=1).reshape(seg, _BATCH)
            x1i = x1_ref[o1 + 16 * i:o1 + 16 * (i + 1), :]
            contrib = x1i[:, None, :] * w[None, :, :]
            acc = contrib if acc is None else acc + contrib
        if acc is None:
            acc = jnp.zeros((16, seg, _BATCH), jnp.float32)
        ob[:, 0:16 * seg] = acc.reshape(16 * seg, _BATCH).T
        cp = pltpu.make_async_copy(
            ob.at[:, 0:256 * K],
            out_ref.at[:, t['soff']:t['soff'] + 256 * K],
            sems.at[buf])
        cp.start()
        pending[buf] = cp
    for p in pending:
        if p is not None:
            p.wait()


@jax.jit
def kernel(in1, in2):
    perm = jnp.asarray(_PERM)
    x1t = in1.T[perm]
    x2t = in2.T[perm]
    return pl.pallas_call(
        _tp_kernel,
        out_shape=jax.ShapeDtypeStruct((_BATCH, _NCOLS), jnp.float32),
        in_specs=[
            pl.BlockSpec(memory_space=pltpu.MemorySpace.VMEM),
            pl.BlockSpec(memory_space=pltpu.MemorySpace.VMEM),
        ],
        out_specs=pl.BlockSpec(memory_space=pltpu.MemorySpace.HBM),
        scratch_shapes=[
            pltpu.VMEM((_BATCH, _MAXSEG), jnp.float32),
            pltpu.VMEM((_BATCH, _MAXSEG), jnp.float32),
            pltpu.SemaphoreType.DMA((2,)),
        ],
    )(x1t, x2t)


# speedup vs baseline: 14.8737x; 1.1345x over previous
import math
from fractions import Fraction

import numpy as np
import jax
import jax.numpy as jnp
from jax.experimental import pallas as pl
from jax.experimental.pallas import tpu as pltpu

_IRREPS = [(16, 0, 1), (16, 1, -1), (16, 2, 1), (16, 3, -1), (16, 4, 1)]
_BATCH = 128
_NCOLS = 160000


def _fact(n):
    return math.factorial(round(n))


def _su2_cg_coeff(j1, m1, j2, m2, j3, m3):
    if m3 != m1 + m2:
        return 0.0
    vmin = int(max(-j1 + j2 + m3, -j1 + m1, 0))
    vmax = int(min(j2 + j3 + m1, j3 - j1 + j2, j3 + m3))
    C = ((2.0 * j3 + 1.0) * Fraction(
        _fact(j3 + j1 - j2) * _fact(j3 - j1 + j2) * _fact(j1 + j2 - j3)
        * _fact(j3 + m3) * _fact(j3 - m3),
        _fact(j1 + j2 + j3 + 1) * _fact(j1 - m1) * _fact(j1 + m1)
        * _fact(j2 - m2) * _fact(j2 + m2))) ** 0.5
    S = 0
    for v in range(vmin, vmax + 1):
        S += (-1) ** int(v + j2 + m2) * Fraction(
            _fact(j2 + j3 + m1 - v) * _fact(j1 - m1 + v),
            _fact(v) * _fact(j3 - j1 + j2 - v) * _fact(j3 + m3 - v)
            * _fact(j1 - j2 - m3 + v))
    return float(C * S)


def _su2_cg(j1, j2, j3):
    mat = np.zeros((2 * j1 + 1, 2 * j2 + 1, 2 * j3 + 1))
    for m1 in range(-j1, j1 + 1):
        for m2 in range(-j2, j2 + 1):
            m3 = m1 + m2
            if abs(m3) <= j3:
                mat[j1 + m1, j2 + m2, j3 + m3] = _su2_cg_coeff(j1, m1, j2, m2, j3, m3)
    return mat


def _real_basis(l):
    q = np.zeros((2 * l + 1, 2 * l + 1), dtype=np.complex128)
    for m in range(-l, 0):
        q[l + m, l + abs(m)] = 1.0 / 2 ** 0.5
        q[l + m, l - abs(m)] = -1j / 2 ** 0.5
    q[l, l] = 1.0
    for m in range(1, l + 1):
        q[l + m, l + abs(m)] = (-1) ** m / 2 ** 0.5
        q[l + m, l - abs(m)] = 1j * (-1) ** m / 2 ** 0.5
    return (-1j) ** l * q


def _w3j(l1, l2, l3):
    C = _su2_cg(l1, l2, l3).astype(np.complex128)
    Q1 = _real_basis(l1)
    Q2 = _real_basis(l2)
    Q3 = _real_basis(l3)
    C = np.einsum('ij,kl,mn,ikn->jlm', Q1, Q2, np.conj(Q3.T), C)
    C = np.real(C)
    n = np.linalg.norm(C)
    if n > 0:
        C = C / n
    return C


def _build_plan():
    offs = []
    o = 0
    for (m, l, p) in _IRREPS:
        offs.append(o)
        o += m * (2 * l + 1)
    gen = []
    for a, (m1, l1, p1) in enumerate(_IRREPS):
        for b, (m2, l2, p2) in enumerate(_IRREPS):
            for l3 in range(abs(l1 - l2), l1 + l2 + 1):
                p3 = int((p1 * p2 + 1) / 2)
                C = (_w3j(l1, l2, l3) * math.sqrt(2 * l3 + 1)).astype(np.float32)
                gen.append(dict(o1=offs[a], o2=offs[b],
                                I=2 * l1 + 1, J=2 * l2 + 1, K=2 * l3 + 1,
                                key=2 * (l3 + 1) + p3, C=C))
    order = sorted(range(len(gen)), key=lambda i: gen[i]['key'])
    soff = 0
    plan = []
    for gi in order:
        t = dict(gen[gi])
        t['soff'] = soff
        soff += 256 * t['K']
        C = t['C']
        inz = []
        for i in range(t['I']):
            inz.append([(j, k, float(C[i, j, k]))
                        for j in range(t['J']) for k in range(t['K'])
                        if abs(C[i, j, k]) > 0.0])
        t['inz'] = inz
        plan.append(t)
    assert soff == _NCOLS
    return plan


_PLAN = _build_plan()
_MAXSEG = 256 * max(t['K'] for t in _PLAN)


def _row_perms():
    perm = []
    o = 0
    for (m, l, p) in _IRREPS:
        d = 2 * l + 1
        for i in range(d):
            for u in range(m):
                perm.append(o + u * d + i)
        o += m * d
    return np.asarray(perm, np.int32)


_PERM = _row_perms()


def _tp_kernel(x1_ref, x2_ref, out_ref, ob0, ob1, sems):
    bufs = (ob0, ob1)
    pending = [None, None]
    for idx, t in enumerate(_PLAN):
        I, J, K = t['I'], t['J'], t['K']
        seg = 16 * K
        buf = idx % 2
        ob = bufs[buf]
        if pending[buf] is not None:
            pending[buf].wait()
            pending[buf] = None
        o1, o2 = t['o1'], t['o2']
        x2j_cache = {}
        acc = None
        for i in range(I):
            if not t['inz'][i]:
                continue
            by_k = {}
            for (j, k, c) in t['inz'][i]:
                if j not in x2j_cache:
                    x2j_cache[j] = x2_ref[o2 + 16 * j:o2 + 16 * (j + 1), :]
                term = x2j_cache[j] * np.float32(c)
                by_k[k] = term if k not in by_k else by_k[k] + term
            wk = [by_k.get(k) for k in range(K)]
            zero = None
            for k in range(K):
                if wk[k] is None:
                    if zero is None:
                        zero = jnp.zeros((16, _BATCH), jnp.float32)
                    wk[k] = zero
            w = jnp.stack(wk, axis=0).transpose(1, 0, 2).reshape(seg, _BATCH)
            x1i = x1_ref[o1 + 16 * i:o1 + 16 * (i + 1), :]
            contrib = x1i[:, None, :] * w[None, :, :]
            acc = contrib if acc is None else acc + contrib
        if acc is None:
            acc = jnp.zeros((16, seg, _BATCH), jnp.float32)
        ob[:, 0:16 * seg] = acc.reshape(16 * seg, _BATCH).T
        cp = pltpu.make_async_copy(
            ob.at[:, 0:256 * K],
            out_ref.at[:, t['soff']:t['soff'] + 256 * K],
            sems.at[buf])
        cp.start()
        pending[buf] = cp
    for p in pending:
        if p is not None:
            p.wait()


@jax.jit
def kernel(in1, in2):
    perm = jnp.asarray(_PERM)
    x1t = in1.T[perm]
    x2t = in2.T[perm]
    return pl.pallas_call(
        _tp_kernel,
        out_shape=jax.ShapeDtypeStruct((_BATCH, _NCOLS), jnp.float32),
        in_specs=[
            pl.BlockSpec(memory_space=pltpu.MemorySpace.VMEM),
            pl.BlockSpec(memory_space=pltpu.MemorySpace.VMEM),
        ],
        out_specs=pl.BlockSpec(memory_space=pltpu.MemorySpace.HBM),
        scratch_shapes=[
            pltpu.VMEM((_BATCH, _MAXSEG), jnp.float32),
            pltpu.VMEM((_BATCH, _MAXSEG), jnp.float32),
            pltpu.SemaphoreType.DMA((2,)),
        ],
    )(x1t, x2t)


# speedup vs baseline: 15.1084x; 1.0158x over previous
import math
from fractions import Fraction

import numpy as np
import jax
import jax.numpy as jnp
from jax.experimental import pallas as pl
from jax.experimental.pallas import tpu as pltpu

_IRREPS = [(16, 0, 1), (16, 1, -1), (16, 2, 1), (16, 3, -1), (16, 4, 1)]
_BATCH = 128
_NCOLS = 160000


def _fact(n):
    return math.factorial(round(n))


def _su2_cg_coeff(j1, m1, j2, m2, j3, m3):
    if m3 != m1 + m2:
        return 0.0
    vmin = int(max(-j1 + j2 + m3, -j1 + m1, 0))
    vmax = int(min(j2 + j3 + m1, j3 - j1 + j2, j3 + m3))
    C = ((2.0 * j3 + 1.0) * Fraction(
        _fact(j3 + j1 - j2) * _fact(j3 - j1 + j2) * _fact(j1 + j2 - j3)
        * _fact(j3 + m3) * _fact(j3 - m3),
        _fact(j1 + j2 + j3 + 1) * _fact(j1 - m1) * _fact(j1 + m1)
        * _fact(j2 - m2) * _fact(j2 + m2))) ** 0.5
    S = 0
    for v in range(vmin, vmax + 1):
        S += (-1) ** int(v + j2 + m2) * Fraction(
            _fact(j2 + j3 + m1 - v) * _fact(j1 - m1 + v),
            _fact(v) * _fact(j3 - j1 + j2 - v) * _fact(j3 + m3 - v)
            * _fact(j1 - j2 - m3 + v))
    return float(C * S)


def _su2_cg(j1, j2, j3):
    mat = np.zeros((2 * j1 + 1, 2 * j2 + 1, 2 * j3 + 1))
    for m1 in range(-j1, j1 + 1):
        for m2 in range(-j2, j2 + 1):
            m3 = m1 + m2
            if abs(m3) <= j3:
                mat[j1 + m1, j2 + m2, j3 + m3] = _su2_cg_coeff(j1, m1, j2, m2, j3, m3)
    return mat


def _real_basis(l):
    q = np.zeros((2 * l + 1, 2 * l + 1), dtype=np.complex128)
    for m in range(-l, 0):
        q[l + m, l + abs(m)] = 1.0 / 2 ** 0.5
        q[l + m, l - abs(m)] = -1j / 2 ** 0.5
    q[l, l] = 1.0
    for m in range(1, l + 1):
        q[l + m, l + abs(m)] = (-1) ** m / 2 ** 0.5
        q[l + m, l - abs(m)] = 1j * (-1) ** m / 2 ** 0.5
    return (-1j) ** l * q


def _w3j(l1, l2, l3):
    C = _su2_cg(l1, l2, l3).astype(np.complex128)
    Q1 = _real_basis(l1)
    Q2 = _real_basis(l2)
    Q3 = _real_basis(l3)
    C = np.einsum('ij,kl,mn,ikn->jlm', Q1, Q2, np.conj(Q3.T), C)
    C = np.real(C)
    n = np.linalg.norm(C)
    if n > 0:
        C = C / n
    return C


def _build_plan():
    offs = []
    o = 0
    for (m, l, p) in _IRREPS:
        offs.append(o)
        o += m * (2 * l + 1)
    gen = []
    for a, (m1, l1, p1) in enumerate(_IRREPS):
        for b, (m2, l2, p2) in enumerate(_IRREPS):
            for l3 in range(abs(l1 - l2), l1 + l2 + 1):
                p3 = int((p1 * p2 + 1) / 2)
                C = (_w3j(l1, l2, l3) * math.sqrt(2 * l3 + 1)).astype(np.float32)
                gen.append(dict(o1=offs[a], o2=offs[b],
                                I=2 * l1 + 1, J=2 * l2 + 1, K=2 * l3 + 1,
                                key=2 * (l3 + 1) + p3, C=C))
    order = sorted(range(len(gen)), key=lambda i: gen[i]['key'])
    soff = 0
    plan = []
    for gi in order:
        t = dict(gen[gi])
        t['soff'] = soff
        soff += 256 * t['K']
        C = t['C']
        inz = []
        for i in range(t['I']):
            inz.append([(j, k, float(C[i, j, k]))
                        for j in range(t['J']) for k in range(t['K'])
                        if abs(C[i, j, k]) > 0.0])
        t['inz'] = inz
        plan.append(t)
    assert soff == _NCOLS
    return plan


_PLAN = _build_plan()
_MAXSEG = 256 * max(t['K'] for t in _PLAN)


def _row_perms():
    perm = []
    o = 0
    for (m, l, p) in _IRREPS:
        d = 2 * l + 1
        for i in range(d):
            for u in range(m):
                perm.append(o + u * d + i)
        o += m * d
    return np.asarray(perm, np.int32)


_PERM = _row_perms()


def _tp_kernel(in1_ref, in2_ref, out_ref, x1_ref, x2_ref, ob0, ob1, sems):
    x1tv = in1_ref[:, :].T
    x2tv = in2_ref[:, :].T
    for r, pr in enumerate(_PERM):
        pr = int(pr)
        x1_ref[r:r + 1, :] = x1tv[pr:pr + 1, :]
        x2_ref[r:r + 1, :] = x2tv[pr:pr + 1, :]
    bufs = (ob0, ob1)
    pending = [None, None]
    for idx, t in enumerate(_PLAN):
        I, J, K = t['I'], t['J'], t['K']
        seg = 16 * K
        buf = idx % 2
        ob = bufs[buf]
        if pending[buf] is not None:
            pending[buf].wait()
            pending[buf] = None
        o1, o2 = t['o1'], t['o2']
        x2j_cache = {}
        acc = None
        for i in range(I):
            if not t['inz'][i]:
                continue
            by_k = {}
            for (j, k, c) in t['inz'][i]:
                if j not in x2j_cache:
                    x2j_cache[j] = x2_ref[o2 + 16 * j:o2 + 16 * (j + 1), :]
                term = x2j_cache[j] * np.float32(c)
                by_k[k] = term if k not in by_k else by_k[k] + term
            wk = [by_k.get(k) for k in range(K)]
            zero = None
            for k in range(K):
                if wk[k] is None:
                    if zero is None:
                        zero = jnp.zeros((16, _BATCH), jnp.float32)
                    wk[k] = zero
            w = jnp.stack(wk, axis=0).transpose(1, 0, 2).reshape(seg, _BATCH)
            x1i = x1_ref[o1 + 16 * i:o1 + 16 * (i + 1), :]
            contrib = x1i[:, None, :] * w[None, :, :]
            acc = contrib if acc is None else acc + contrib
        if acc is None:
            acc = jnp.zeros((16, seg, _BATCH), jnp.float32)
        ob[:, 0:16 * seg] = acc.reshape(16 * seg, _BATCH).T
        cp = pltpu.make_async_copy(
            ob.at[:, 0:256 * K],
            out_ref.at[:, t['soff']:t['soff'] + 256 * K],
            sems.at[buf])
        cp.start()
        pending[buf] = cp
    for p in pending:
        if p is not None:
            p.wait()


@jax.jit
def kernel(in1, in2):
    return pl.pallas_call(
        _tp_kernel,
        out_shape=jax.ShapeDtypeStruct((_BATCH, _NCOLS), jnp.float32),
        in_specs=[
            pl.BlockSpec(memory_space=pltpu.MemorySpace.VMEM),
            pl.BlockSpec(memory_space=pltpu.MemorySpace.VMEM),
        ],
        out_specs=pl.BlockSpec(memory_space=pltpu.MemorySpace.HBM),
        scratch_shapes=[
            pltpu.VMEM((400, _BATCH), jnp.float32),
            pltpu.VMEM((400, _BATCH), jnp.float32),
            pltpu.VMEM((_BATCH, _MAXSEG), jnp.float32),
            pltpu.VMEM((_BATCH, _MAXSEG), jnp.float32),
            pltpu.SemaphoreType.DMA((2,)),
        ],
    )(in1, in2)


# speedup vs baseline: 17.7370x; 1.1740x over previous
import math
from fractions import Fraction

import numpy as np
import jax
import jax.numpy as jnp
from jax.experimental import pallas as pl
from jax.experimental.pallas import tpu as pltpu

_IRREPS = [(16, 0, 1), (16, 1, -1), (16, 2, 1), (16, 3, -1), (16, 4, 1)]
_BATCH = 128
_NCOLS = 160000


def _fact(n):
    return math.factorial(round(n))


def _su2_cg_coeff(j1, m1, j2, m2, j3, m3):
    if m3 != m1 + m2:
        return 0.0
    vmin = int(max(-j1 + j2 + m3, -j1 + m1, 0))
    vmax = int(min(j2 + j3 + m1, j3 - j1 + j2, j3 + m3))
    C = ((2.0 * j3 + 1.0) * Fraction(
        _fact(j3 + j1 - j2) * _fact(j3 - j1 + j2) * _fact(j1 + j2 - j3)
        * _fact(j3 + m3) * _fact(j3 - m3),
        _fact(j1 + j2 + j3 + 1) * _fact(j1 - m1) * _fact(j1 + m1)
        * _fact(j2 - m2) * _fact(j2 + m2))) ** 0.5
    S = 0
    for v in range(vmin, vmax + 1):
        S += (-1) ** int(v + j2 + m2) * Fraction(
            _fact(j2 + j3 + m1 - v) * _fact(j1 - m1 + v),
            _fact(v) * _fact(j3 - j1 + j2 - v) * _fact(j3 + m3 - v)
            * _fact(j1 - j2 - m3 + v))
    return float(C * S)


def _su2_cg(j1, j2, j3):
    mat = np.zeros((2 * j1 + 1, 2 * j2 + 1, 2 * j3 + 1))
    for m1 in range(-j1, j1 + 1):
        for m2 in range(-j2, j2 + 1):
            m3 = m1 + m2
            if abs(m3) <= j3:
                mat[j1 + m1, j2 + m2, j3 + m3] = _su2_cg_coeff(j1, m1, j2, m2, j3, m3)
    return mat


def _real_basis(l):
    q = np.zeros((2 * l + 1, 2 * l + 1), dtype=np.complex128)
    for m in range(-l, 0):
        q[l + m, l + abs(m)] = 1.0 / 2 ** 0.5
        q[l + m, l - abs(m)] = -1j / 2 ** 0.5
    q[l, l] = 1.0
    for m in range(1, l + 1):
        q[l + m, l + abs(m)] = (-1) ** m / 2 ** 0.5
        q[l + m, l - abs(m)] = 1j * (-1) ** m / 2 ** 0.5
    return (-1j) ** l * q


def _w3j(l1, l2, l3):
    C = _su2_cg(l1, l2, l3).astype(np.complex128)
    Q1 = _real_basis(l1)
    Q2 = _real_basis(l2)
    Q3 = _real_basis(l3)
    C = np.einsum('ij,kl,mn,ikn->jlm', Q1, Q2, np.conj(Q3.T), C)
    C = np.real(C)
    n = np.linalg.norm(C)
    if n > 0:
        C = C / n
    return C


def _build_plan():
    offs = []
    o = 0
    for (m, l, p) in _IRREPS:
        offs.append(o)
        o += m * (2 * l + 1)
    gen = []
    for a, (m1, l1, p1) in enumerate(_IRREPS):
        for b, (m2, l2, p2) in enumerate(_IRREPS):
            for l3 in range(abs(l1 - l2), l1 + l2 + 1):
                p3 = int((p1 * p2 + 1) / 2)
                C = (_w3j(l1, l2, l3) * math.sqrt(2 * l3 + 1)).astype(np.float32)
                gen.append(dict(o1=offs[a], o2=offs[b],
                                I=2 * l1 + 1, J=2 * l2 + 1, K=2 * l3 + 1,
                                key=2 * (l3 + 1) + p3, C=C))
    order = sorted(range(len(gen)), key=lambda i: gen[i]['key'])
    soff = 0
    plan = []
    for gi in order:
        t = dict(gen[gi])
        t['soff'] = soff
        soff += 256 * t['K']
        C = t['C']
        inz = []
        for i in range(t['I']):
            inz.append([(j, k, float(C[i, j, k]))
                        for j in range(t['J']) for k in range(t['K'])
                        if abs(C[i, j, k]) > 0.0])
        t['inz'] = inz
        plan.append(t)
    assert soff == _NCOLS
    return plan


_PLAN = _build_plan()
_MAXSEG = 256 * max(t['K'] for t in _PLAN)
_OBW = 8192


def _row_perms():
    perm = []
    o = 0
    for (m, l, p) in _IRREPS:
        d = 2 * l + 1
        for i in range(d):
            for u in range(m):
                perm.append(o + u * d + i)
        o += m * d
    return np.asarray(perm, np.int32)


_PERM = _row_perms()


def _tp_kernel(in1_ref, in2_ref, out_ref, x1_ref, x2_ref, ob0, ob1, sems):
    x1tv = in1_ref[:, :].T
    x2tv = in2_ref[:, :].T
    for r, pr in enumerate(_PERM):
        pr = int(pr)
        x1_ref[r:r + 1, :] = x1tv[pr:pr + 1, :]
        x2_ref[r:r + 1, :] = x2tv[pr:pr + 1, :]
    bufs = (ob0, ob1)
    pending = [None, None]
    buf = 0
    cur = 0
    flush_off = 0
    for idx, t in enumerate(_PLAN):
        I, J, K = t['I'], t['J'], t['K']
        seg = 16 * K
        if cur + 256 * K > _OBW:
            cp = pltpu.make_async_copy(
                bufs[buf].at[:, 0:cur],
                out_ref.at[:, flush_off:flush_off + cur],
                sems.at[buf])
            cp.start()
            pending[buf] = cp
            flush_off += cur
            cur = 0
            buf ^= 1
            if pending[buf] is not None:
                pending[buf].wait()
                pending[buf] = None
        ob = bufs[buf]
        o1, o2 = t['o1'], t['o2']
        x2j_cache = {}
        acc = None
        for i in range(I):
            if not t['inz'][i]:
                continue
            by_k = {}
            for (j, k, c) in t['inz'][i]:
                if j not in x2j_cache:
                    x2j_cache[j] = x2_ref[o2 + 16 * j:o2 + 16 * (j + 1), :]
                term = x2j_cache[j] * np.float32(c)
                by_k[k] = term if k not in by_k else by_k[k] + term
            wk = [by_k.get(k) for k in range(K)]
            zero = None
            for k in range(K):
                if wk[k] is None:
                    if zero is None:
                        zero = jnp.zeros((16, _BATCH), jnp.float32)
                    wk[k] = zero
            w = jnp.stack(wk, axis=0).transpose(1, 0, 2).reshape(seg, _BATCH)
            x1i = x1_ref[o1 + 16 * i:o1 + 16 * (i + 1), :]
            contrib = x1i[:, None, :] * w[None, :, :]
            acc = contrib if acc is None else acc + contrib
        if acc is None:
            acc = jnp.zeros((16, seg, _BATCH), jnp.float32)
        ob[:, cur:cur + 16 * seg] = acc.reshape(16 * seg, _BATCH).T
        cur += 256 * K
    if cur:
        cp = pltpu.make_async_copy(
            bufs[buf].at[:, 0:cur],
            out_ref.at[:, flush_off:flush_off + cur],
            sems.at[buf])
        cp.start()
        pending[buf] = cp
    for p in pending:
        if p is not None:
            p.wait()


@jax.jit
def kernel(in1, in2):
    return pl.pallas_call(
        _tp_kernel,
        out_shape=jax.ShapeDtypeStruct((_BATCH, _NCOLS), jnp.float32),
        in_specs=[
            pl.BlockSpec(memory_space=pltpu.MemorySpace.VMEM),
            pl.BlockSpec(memory_space=pltpu.MemorySpace.VMEM),
        ],
        out_specs=pl.BlockSpec(memory_space=pltpu.MemorySpace.HBM),
        scratch_shapes=[
            pltpu.VMEM((400, _BATCH), jnp.float32),
            pltpu.VMEM((400, _BATCH), jnp.float32),
            pltpu.VMEM((_BATCH, _OBW), jnp.float32),
            pltpu.VMEM((_BATCH, _OBW), jnp.float32),
            pltpu.SemaphoreType.DMA((2,)),
        ],
    )(in1, in2)
